# baseline (speedup 1.0000x reference)
import jax
import jax.numpy as jnp
from jax import lax
from jax.experimental import pallas as pl
from jax.experimental.pallas import tpu as pltpu
from jax.experimental.pallas import tpu_sc as plsc

DIM = 64

_info = plsc.get_sparse_core_info()
NC, NS = _info.num_cores, _info.num_subcores
NW = NC * NS

NI = 4096
NJ = 200
IB = NI // NW
NCT = NI // 128


def _do_transpose(rows, tb, b, row_ids, zero16):
    for d in range(DIM):
        a, bb = d // 8, d % 8
        for g in range(8):
            v = plsc.load_gather(rows.at[b], [row_ids[g], zero16 + d])
            tb[b, a, pl.ds(128 * bb + 16 * g, 16)] = v


def _gather_body(idx_hbm, table_hbm, out_hbm, idx_v, rows, tb, gsem, wsem):
    wid = lax.axis_index("s") * NC + lax.axis_index("c")
    pltpu.sync_copy(idx_hbm.at[(pl.ds(0, NJ), pl.ds(wid * IB, IB))], idx_v)

    iota = lax.iota(jnp.int32, 16)
    row_ids = [iota + (16 * g) for g in range(8)]
    zero16 = iota * 0

    def fire_gather(j, b):
        pltpu.async_copy(table_hbm.at[idx_v.at[j]], rows.at[b], gsem.at[b])

    def wait_gather(b):
        pltpu.make_async_copy(
            table_hbm.at[idx_v.at[0]], rows.at[b], gsem.at[b]
        ).wait()

    def fire_write(j, b):
        pltpu.async_copy(
            tb.at[b], out_hbm.at[(j, pl.ds(0, 8), wid)], wsem.at[b]
        )

    def wait_write(b):
        pltpu.make_async_copy(
            tb.at[b], out_hbm.at[(0, pl.ds(0, 8), 0)], wsem.at[b]
        ).wait()

    fire_gather(0, 0)
    fire_gather(1, 1)
    for j in (0, 1):
        wait_gather(j % 2)
        _do_transpose(rows, tb, j % 2, row_ids, zero16)
        fire_write(j, j % 2)
        fire_gather(j + 2, j % 2)

    def steady(it, _):
        j0 = 2 + it * 2
        for u in range(2):
            j = j0 + u
            wait_gather(u)
            wait_write(u)
            _do_transpose(rows, tb, u, row_ids, zero16)
            fire_write(j, u)
            fire_gather(j + 2, u)
        return _

    lax.fori_loop(0, (NJ - 4) // 2, steady, None)

    for j in (NJ - 2, NJ - 1):
        b = j % 2
        wait_gather(b)
        wait_write(b)
        _do_transpose(rows, tb, b, row_ids, zero16)
        fire_write(j, b)
    for b in (0, 1):
        wait_write(b)


@jax.jit
def _embed_gather(idx2, table_pad):
    mesh = plsc.VectorSubcoreMesh(core_axis_name="c", subcore_axis_name="s")
    run = pl.kernel(
        _gather_body,
        mesh=mesh,
        out_type=jax.ShapeDtypeStruct((NJ, 8, NCT, 1024), jnp.float32),
        scratch_types=[
            pltpu.VMEM((NJ, IB), jnp.int32),
            pltpu.VMEM((2, IB, DIM), jnp.float32),
            pltpu.VMEM((2, 8, 1024), jnp.float32),
            pltpu.SemaphoreType.DMA((2,)),
            pltpu.SemaphoreType.DMA((2,)),
        ],
        compiler_params=pltpu.CompilerParams(
            use_tc_tiling_on_sc=False, needs_layout_passes=False
        ),
    )
    return run(idx2, table_pad)


def kernel(x, table):
    idx2 = x.T.astype(jnp.int32) * 2
    table_pad = jnp.pad(table, ((0, 0), (0, DIM))).reshape(2 * 1000000, DIM)
    u = _embed_gather(idx2, table_pad)
    out = (
        u.reshape(NJ, 8, NCT, 8, 128)
        .transpose(2, 4, 0, 1, 3)
        .reshape(NI, NJ, DIM)
    )
    return out


# speedup vs baseline: 2.2659x; 2.2659x over previous
import jax
import jax.numpy as jnp
from jax import lax
from jax.experimental import pallas as pl
from jax.experimental.pallas import tpu as pltpu
from jax.experimental.pallas import tpu_sc as plsc

DIM = 64

_info = plsc.get_sparse_core_info()
NC, NS = _info.num_cores, _info.num_subcores
NW = NC * NS

B_TOTAL = 4096 * 200
ROWS_PER_W = B_TOTAL // NW
SLAB = 128
N_SLABS = ROWS_PER_W // SLAB
NB = 10
LAG = 5


def _gather_body(idx_hbm, table_hbm, out_hbm, idx2_v, rows, gsem, wsem):
    wid = lax.axis_index("s") * NC + lax.axis_index("c")
    base = wid * ROWS_PER_W
    pltpu.sync_copy(idx_hbm.at[pl.ds(base, ROWS_PER_W)], idx2_v)
    for k in range(ROWS_PER_W // 16):
        idx2_v[pl.ds(k * 16, 16)] = idx2_v[pl.ds(k * 16, 16)] * 2

    def fire_gather(s, b):
        pltpu.async_copy(
            table_hbm.at[idx2_v.at[pl.ds(s * SLAB, SLAB)]],
            rows.at[b],
            gsem.at[b],
        )

    def wait_gather(b):
        pltpu.make_async_copy(
            table_hbm.at[idx2_v.at[pl.ds(0, SLAB)]], rows.at[b], gsem.at[b]
        ).wait()

    def fire_write(s, b):
        pltpu.async_copy(
            rows.at[b],
            out_hbm.at[(pl.ds(base + s * SLAB, SLAB), pl.ds(0, DIM))],
            wsem.at[b],
        )

    def wait_write(b):
        pltpu.make_async_copy(
            rows.at[b], out_hbm.at[(pl.ds(0, SLAB), pl.ds(0, DIM))], wsem.at[b]
        ).wait()

    for s in range(LAG):
        fire_gather(s, s % NB)
    for s in range(LAG, NB):
        fire_gather(s, s % NB)
        t = s - LAG
        wait_gather(t % NB)
        fire_write(t, t % NB)

    G = (N_SLABS - NB) // NB
    def steady(g, _):
        s0 = NB + g * NB
        for u in range(NB):
            s = s0 + u
            wait_write(u)
            fire_gather(s, u)
            t = s - LAG
            wait_gather((u + LAG) % NB)
            fire_write(t, (u + LAG) % NB)
        return _

    lax.fori_loop(0, G, steady, None)

    for t in range(N_SLABS - LAG, N_SLABS):
        wait_gather(t % NB)
        fire_write(t, t % NB)
    for b in range(NB):
        wait_write(b)


@jax.jit
def _embed_gather(x_flat, table_pad):
    mesh = plsc.VectorSubcoreMesh(core_axis_name="c", subcore_axis_name="s")
    run = pl.kernel(
        _gather_body,
        mesh=mesh,
        out_type=jax.ShapeDtypeStruct((B_TOTAL, 2 * DIM), jnp.float32),
        scratch_types=[
            pltpu.VMEM((ROWS_PER_W,), jnp.int32),
            pltpu.VMEM((NB, SLAB, DIM), jnp.float32),
            pltpu.SemaphoreType.DMA((NB,)),
            pltpu.SemaphoreType.DMA((NB,)),
        ],
        compiler_params=pltpu.CompilerParams(use_tc_tiling_on_sc=False),
    )
    return run(x_flat, table_pad)


def kernel(x, table):
    x_flat = x.reshape(B_TOTAL).astype(jnp.int32)
    table_pad = jnp.pad(table, ((0, 0), (0, DIM))).reshape(2 * 1000000, DIM)
    out = _embed_gather(x_flat, table_pad)
    return out.reshape(x.shape[0], x.shape[1], 2 * DIM)[:, :, :DIM]
